# baseline (speedup 1.0000x reference)
import functools

import jax
import jax.numpy as jnp
from jax.experimental import pallas as pl
from jax.experimental.pallas import tpu as pltpu

_N_EXPERTS = 64
_TOP_K = 8
_TILE = 512


def _gate_kernel(x_ref, w_ref, idx_ref, wgt_ref):
    x = x_ref[...]
    w = w_ref[...]
    logits = jax.lax.dot_general(
        x, w, (((1,), (1,)), ((), ())), preferred_element_type=jnp.float32
    )
    t = logits.shape[0]
    rev_lane = (
        jnp.int32(_N_EXPERTS - 1)
        - jax.lax.broadcasted_iota(jnp.int32, (t, _N_EXPERTS), 1)
    ).astype(jnp.float32)
    rm = jnp.max(logits, axis=1, keepdims=True)
    p = jnp.exp(logits - rm)
    vals = []
    lanes = []
    for _ in range(_TOP_K):
        m = jnp.max(p, axis=1, keepdims=True)
        r = jnp.max(jnp.where(p == m, rev_lane, -1.0), axis=1, keepdims=True)
        vals.append(m)
        lanes.append(r)
        p = jnp.where(rev_lane == r, -1.0, p)
    e = jnp.concatenate(vals, axis=1)
    r8 = jnp.concatenate(lanes, axis=1)
    idx_ref[...] = jnp.int32(_N_EXPERTS - 1) - r8.astype(jnp.int32)
    wgt_ref[...] = e / jnp.sum(e, axis=1, keepdims=True)


@functools.partial(jax.jit, static_argnums=())
def kernel(hidden_states, weight):
    bsz, seq, h = hidden_states.shape
    tokens = bsz * seq
    x = hidden_states.reshape(tokens, h).astype(jnp.float32)
    w = weight.astype(jnp.float32)
    grid = (tokens // _TILE,)
    idx, wgt = pl.pallas_call(
        _gate_kernel,
        grid=grid,
        in_specs=[
            pl.BlockSpec((_TILE, h), lambda i: (i, 0)),
            pl.BlockSpec((_N_EXPERTS, h), lambda i: (0, 0)),
        ],
        out_specs=[
            pl.BlockSpec((_TILE, _TOP_K), lambda i: (i, 0)),
            pl.BlockSpec((_TILE, _TOP_K), lambda i: (i, 0)),
        ],
        out_shape=[
            jax.ShapeDtypeStruct((tokens, _TOP_K), jnp.int32),
            jax.ShapeDtypeStruct((tokens, _TOP_K), jnp.float32),
        ],
        compiler_params=pltpu.CompilerParams(
            dimension_semantics=("parallel",)
        ),
    )(x, w)
    return idx, wgt


# speedup vs baseline: 1.1126x; 1.1126x over previous
import functools

import jax
import jax.numpy as jnp
from jax.experimental import pallas as pl
from jax.experimental.pallas import tpu as pltpu

_N_EXPERTS = 64
_TOP_K = 8
_TILE = 512


def _gate_kernel(x_ref, w_ref, idx_ref, wgt_ref):
    x = x_ref[...]
    w = w_ref[...]
    logits = jax.lax.dot_general(
        w, x, (((1,), (1,)), ((), ())), preferred_element_type=jnp.float32
    )
    t = logits.shape[1]
    rev = (
        jnp.int32(_N_EXPERTS - 1)
        - jax.lax.broadcasted_iota(jnp.int32, (_N_EXPERTS, t), 0)
    ).astype(jnp.float32)
    rm = jnp.max(logits, axis=0, keepdims=True)
    p = jnp.exp(logits - rm)
    vals = []
    lanes = []
    for _ in range(_TOP_K):
        m = jnp.max(p, axis=0, keepdims=True)
        r = jnp.max(jnp.where(p == m, rev, -1.0), axis=0, keepdims=True)
        vals.append(m)
        lanes.append(r)
        p = jnp.where(rev == r, -1.0, p)
    e = jnp.concatenate(vals, axis=0)
    r8 = jnp.concatenate(lanes, axis=0)
    idx = jnp.int32(_N_EXPERTS - 1) - r8.astype(jnp.int32)
    wgt = e / jnp.sum(e, axis=0, keepdims=True)
    idx_ref[...] = idx.T
    wgt_ref[...] = wgt.T


@functools.partial(jax.jit, static_argnums=())
def kernel(hidden_states, weight):
    bsz, seq, h = hidden_states.shape
    tokens = bsz * seq
    x = hidden_states.reshape(tokens, h).astype(jnp.float32)
    w = weight.astype(jnp.float32)
    grid = (tokens // _TILE,)
    idx, wgt = pl.pallas_call(
        _gate_kernel,
        grid=grid,
        in_specs=[
            pl.BlockSpec((_TILE, h), lambda i: (i, 0)),
            pl.BlockSpec((_N_EXPERTS, h), lambda i: (0, 0)),
        ],
        out_specs=[
            pl.BlockSpec((_TILE, _TOP_K), lambda i: (i, 0)),
            pl.BlockSpec((_TILE, _TOP_K), lambda i: (i, 0)),
        ],
        out_shape=[
            jax.ShapeDtypeStruct((tokens, _TOP_K), jnp.int32),
            jax.ShapeDtypeStruct((tokens, _TOP_K), jnp.float32),
        ],
        compiler_params=pltpu.CompilerParams(
            dimension_semantics=("parallel",)
        ),
    )(x, w)
    return idx, wgt


# speedup vs baseline: 1.1536x; 1.0369x over previous
import functools

import jax
import jax.numpy as jnp
from jax.experimental import pallas as pl
from jax.experimental.pallas import tpu as pltpu

_N_EXPERTS = 64
_TOP_K = 8
_TILE = 1024


def _gate_kernel(x_ref, w_ref, idx_ref, wgt_ref):
    x = x_ref[...]
    w = w_ref[...]
    logits = jax.lax.dot_general(
        w, x, (((1,), (1,)), ((), ())), preferred_element_type=jnp.float32
    )
    t = logits.shape[1]
    rev = (
        jnp.int32(_N_EXPERTS - 1)
        - jax.lax.broadcasted_iota(jnp.int32, (_N_EXPERTS, t), 0)
    ).astype(jnp.float32)
    rm = jnp.max(logits, axis=0, keepdims=True)
    p = jnp.exp(logits - rm)
    vals = []
    lanes = []
    for _ in range(_TOP_K):
        m = jnp.max(p, axis=0, keepdims=True)
        r = jnp.max(jnp.where(p == m, rev, -1.0), axis=0, keepdims=True)
        vals.append(m)
        lanes.append(r)
        p = jnp.where(rev == r, -1.0, p)
    e = jnp.concatenate(vals, axis=0)
    r8 = jnp.concatenate(lanes, axis=0)
    idx = jnp.int32(_N_EXPERTS - 1) - r8.astype(jnp.int32)
    wgt = e / jnp.sum(e, axis=0, keepdims=True)
    idx_ref[...] = idx.T
    wgt_ref[...] = wgt.T


@functools.partial(jax.jit, static_argnums=())
def kernel(hidden_states, weight):
    bsz, seq, h = hidden_states.shape
    tokens = bsz * seq
    x = hidden_states.reshape(tokens, h).astype(jnp.float32)
    w = weight.astype(jnp.float32)
    grid = (tokens // _TILE,)
    idx, wgt = pl.pallas_call(
        _gate_kernel,
        grid=grid,
        in_specs=[
            pl.BlockSpec((_TILE, h), lambda i: (i, 0)),
            pl.BlockSpec((_N_EXPERTS, h), lambda i: (0, 0)),
        ],
        out_specs=[
            pl.BlockSpec((_TILE, _TOP_K), lambda i: (i, 0)),
            pl.BlockSpec((_TILE, _TOP_K), lambda i: (i, 0)),
        ],
        out_shape=[
            jax.ShapeDtypeStruct((tokens, _TOP_K), jnp.int32),
            jax.ShapeDtypeStruct((tokens, _TOP_K), jnp.float32),
        ],
        compiler_params=pltpu.CompilerParams(
            dimension_semantics=("parallel",)
        ),
    )(x, w)
    return idx, wgt
